# baseline (speedup 1.0000x reference)
import functools
import math

import jax
import jax.numpy as jnp
from jax import lax
from jax.experimental import pallas as pl
from jax.experimental.pallas import tpu as pltpu
from jax.experimental.pallas import tpu_sc as plsc

N_NODES = 10000
N_EDGES = 320000
D_FEAT = 128
D_ATTR = 4
D_RADIAL = 8
D_HID = 64
AVG_NUM_NEIGHBORS = 32.0

_NC = 2
_NS = 16
_LANES = 16

_C = 80
_EPT = N_EDGES // (_NC * _NS)
_STEPS = _EPT // _C
_NPAD = 10240
_RPT = _NPAD // _NS


def _h_body(nf_ref, w_ref, o_ref):
    o_ref[...] = jnp.dot(nf_ref[...], w_ref[...],
                         preferred_element_type=jnp.float32) * (1.0 / math.sqrt(D_FEAT))


def _compute_h(node_feats, W_up):
    blk = 2000
    grid = N_NODES // blk
    return pl.pallas_call(
        _h_body,
        grid=(grid,),
        in_specs=[
            pl.BlockSpec((blk, D_FEAT), lambda i: (i, 0)),
            pl.BlockSpec((D_FEAT, D_FEAT), lambda i: (0, 0)),
        ],
        out_specs=pl.BlockSpec((blk, D_FEAT), lambda i: (i, 0)),
        out_shape=jax.ShapeDtypeStruct((N_NODES, D_FEAT), jnp.float32),
    )(node_feats, W_up)


def _q_body(ef_ref, ea_ref, w1_ref, w2_ref, w3_ref, w4_ref, o_ref):
    t = jax.nn.silu(
        lax.dot_general(ef_ref[...], w1_ref[...], (((0,), (0,)), ((), ())),
                        preferred_element_type=jnp.float32) * (1.0 / math.sqrt(D_RADIAL)))
    t = jax.nn.silu(jnp.dot(t, w2_ref[...],
                            preferred_element_type=jnp.float32) * (1.0 / math.sqrt(D_HID)))
    t = jax.nn.silu(jnp.dot(t, w3_ref[...],
                            preferred_element_type=jnp.float32) * (1.0 / math.sqrt(D_HID)))
    w = jnp.dot(t, w4_ref[...], preferred_element_type=jnp.float32) * (1.0 / math.sqrt(D_HID))
    o_ref[...] = w * lax.transpose(ea_ref[...], (1, 0))


def _compute_q(ef_t, ea_t, W1, W2, W3, W4, col0, nrows):
    blk = 6400
    grid = nrows // blk
    off = col0 // blk
    return pl.pallas_call(
        _q_body,
        grid=(grid,),
        in_specs=[
            pl.BlockSpec((D_RADIAL, blk), lambda i: (0, i + off)),
            pl.BlockSpec((1, blk), lambda i: (0, i + off)),
            pl.BlockSpec((D_RADIAL, D_HID), lambda i: (0, 0)),
            pl.BlockSpec((D_HID, D_HID), lambda i: (0, 0)),
            pl.BlockSpec((D_HID, D_HID), lambda i: (0, 0)),
            pl.BlockSpec((D_HID, D_FEAT), lambda i: (0, 0)),
        ],
        out_specs=pl.BlockSpec((blk, D_FEAT), lambda i: (i, 0)),
        out_shape=jax.ShapeDtypeStruct((nrows, D_FEAT), jnp.float32),
    )(ef_t, ea_t, W1, W2, W3, W4)


def _make_sc_kernel(ept, steps, c):
    mesh = plsc.VectorSubcoreMesh(core_axis_name="c", subcore_axis_name="s")

    def body_fn(idx_hbm, q_hbm, h_hbm, out_hbm,
                idx_v, q_v, g_v, acc, isem0, isem1, isem2, isem3,
                qsem, gsem, ssem):
        ci = lax.axis_index("c")
        s = lax.axis_index("s")
        wid = s * _NC + ci
        isems = (isem0, isem1, isem2, isem3)

        def idxload(k, ib):
            pltpu.async_copy(idx_hbm.at[wid, k], idx_v.at[ib], isems[ib])

        def wait_idx(ib):
            pltpu.make_async_copy(idx_hbm.at[0, 0], idx_v.at[ib], isems[ib]).wait()

        def qgload(k, qb, ib):
            base = wid * ept + k * c
            pltpu.async_copy(q_hbm.at[pl.ds(base, c), :], q_v.at[qb], qsem)
            pltpu.async_copy(h_hbm.at[idx_v.at[ib, 0]], g_v.at[qb], gsem)

        def wait_qg(qb, ib):
            pltpu.make_async_copy(q_hbm.at[pl.ds(0, c), :], q_v.at[qb], qsem).wait()
            pltpu.make_async_copy(h_hbm.at[idx_v.at[ib, 0]], g_v.at[qb], gsem).wait()

        def scatter(qb, ib):
            pltpu.async_copy(q_v.at[qb], acc.at[idx_v.at[ib, 1]], ssem, add=True)

        def wait_scatter(qb, ib):
            pltpu.make_async_copy(q_v.at[qb], acc.at[idx_v.at[ib, 1]], ssem).wait()

        def mul_chunk(qb):
            def mbody(e, _):
                for j in range(D_FEAT // _LANES):
                    sl = pl.ds(j * _LANES, _LANES)
                    q_v[qb, e, sl] = q_v[qb, e, sl] * g_v[qb, e, sl]
                return 0
            lax.fori_loop(0, c, mbody, 0)

        idxload(0, 0)
        idxload(1, 1)
        def zbody(e, _):
            for j in range(D_FEAT // _LANES):
                g_v[0, e, pl.ds(j * _LANES, _LANES)] = jnp.zeros((_LANES,), jnp.float32)
            return 0
        lax.fori_loop(0, c, zbody, 0)
        row0 = s * _RPT
        for m in range(_RPT // c):
            pltpu.sync_copy(g_v.at[0], acc.at[pl.ds(row0 + m * c, c), :])
        plsc.subcore_barrier()

        wait_idx(0)
        qgload(0, 0, 0)

        def body(k, qb, ib):
            wait_qg(qb, ib)
            lax.cond(k + 2 < steps,
                     lambda: idxload(k + 2, (ib + 2) % 4), lambda: None)

            def prep_next():
                lax.cond(k >= 1, lambda: wait_scatter(1 - qb, (ib + 3) % 4),
                         lambda: None)
                wait_idx((ib + 1) % 4)
                qgload(k + 1, 1 - qb, (ib + 1) % 4)
            lax.cond(k + 1 < steps, prep_next, lambda: None)

            mul_chunk(qb)
            scatter(qb, ib)

        def step4(i, _):
            for m in range(4):
                body(4 * i + m, m % 2, m)
            return 0
        lax.fori_loop(0, steps // 4, step4, 0)
        for t in range(steps - (steps // 4) * 4):
            kt = (steps // 4) * 4 + t
            body(kt, kt % 2, kt % 4)
        wait_scatter(0, 0)
        wait_scatter(0, 0)

        plsc.subcore_barrier()

        for m in range(_RPT // c):
            r0 = row0 + m * c
            pltpu.sync_copy(acc.at[pl.ds(r0, c), :], g_v.at[0])
            pltpu.sync_copy(g_v.at[0], out_hbm.at[ci, pl.ds(r0, c), :])

    return pl.kernel(
        body_fn,
        out_type=jax.ShapeDtypeStruct((_NC, _NPAD, D_FEAT), jnp.float32),
        mesh=mesh,
        scratch_types=[
            pltpu.VMEM((4, 2, c), jnp.int32),
            pltpu.VMEM((2, c, D_FEAT), jnp.float32),
            pltpu.VMEM((2, c, D_FEAT), jnp.float32),
            pltpu.VMEM_SHARED((_NPAD, D_FEAT), jnp.float32),
            pltpu.SemaphoreType.DMA,
            pltpu.SemaphoreType.DMA,
            pltpu.SemaphoreType.DMA,
            pltpu.SemaphoreType.DMA,
            pltpu.SemaphoreType.DMA,
            pltpu.SemaphoreType.DMA,
            pltpu.SemaphoreType.DMA,
        ],
    )


def _scatter_gather(sender, receiver, q, h, ept, steps, c):
    f = _make_sc_kernel(ept, steps, c)
    idx = jnp.stack([sender.reshape(_NC * _NS, steps, c),
                     receiver.reshape(_NC * _NS, steps, c)], axis=2)
    return f(idx, q, h)


def _fin_body(pa_ref, pb_ref, w_ref, o_ref):
    scale = 1.0 / math.sqrt(D_FEAT) / AVG_NUM_NEIGHBORS
    msg = pa_ref[0] + pa_ref[1] + pb_ref[0] + pb_ref[1]
    o_ref[...] = jnp.dot(msg, w_ref[...], preferred_element_type=jnp.float32) * scale


def _finalize(pa, pb, W_lin):
    blk = 1000
    grid = N_NODES // blk
    return pl.pallas_call(
        _fin_body,
        grid=(grid,),
        in_specs=[
            pl.BlockSpec((_NC, blk, D_FEAT), lambda i: (0, i, 0)),
            pl.BlockSpec((_NC, blk, D_FEAT), lambda i: (0, i, 0)),
            pl.BlockSpec((D_FEAT, D_FEAT), lambda i: (0, 0)),
        ],
        out_specs=pl.BlockSpec((blk, D_FEAT), lambda i: (i, 0)),
        out_shape=jax.ShapeDtypeStruct((N_NODES, D_FEAT), jnp.float32),
    )(pa, pb, W_lin)


def kernel(node_attrs, node_feats, edge_attrs, edge_feats, edge_index,
           W_up, W1, W2, W3, W4, W_lin, W_skip):
    sender = edge_index[0].astype(jnp.int32)
    receiver = edge_index[1].astype(jnp.int32)
    ef_t = edge_feats.T
    ea_t = edge_attrs.reshape(1, N_EDGES)
    e2 = N_EDGES // 2
    ept = e2 // (_NC * _NS)
    c2 = 40
    steps = ept // c2
    h = _compute_h(node_feats, W_up)
    q_a = _compute_q(ef_t, ea_t, W1, W2, W3, W4, 0, e2)
    p_a = _scatter_gather(sender[:e2], receiver[:e2], q_a, h, ept, steps, c2)
    q_b = _compute_q(ef_t, ea_t, W1, W2, W3, W4, e2, e2)
    p_b = _scatter_gather(sender[e2:], receiver[e2:], q_b, h, ept, steps, c2)
    return _finalize(p_a, p_b, W_lin)


# speedup vs baseline: 1.0029x; 1.0029x over previous
import functools
import math

import jax
import jax.numpy as jnp
from jax import lax
from jax.experimental import pallas as pl
from jax.experimental.pallas import tpu as pltpu
from jax.experimental.pallas import tpu_sc as plsc

N_NODES = 10000
N_EDGES = 320000
D_FEAT = 128
D_ATTR = 4
D_RADIAL = 8
D_HID = 64
AVG_NUM_NEIGHBORS = 32.0

_NC = 2
_NS = 16
_LANES = 16

_C = 80
_EPT = N_EDGES // (_NC * _NS)
_STEPS = _EPT // _C
_NPAD = 10240
_RPT = _NPAD // _NS


def _h_body(nf_ref, w_ref, o_ref):
    o_ref[...] = jnp.dot(nf_ref[...], w_ref[...],
                         preferred_element_type=jnp.float32) * (1.0 / math.sqrt(D_FEAT))


def _compute_h(node_feats, W_up):
    blk = 2000
    grid = N_NODES // blk
    return pl.pallas_call(
        _h_body,
        grid=(grid,),
        in_specs=[
            pl.BlockSpec((blk, D_FEAT), lambda i: (i, 0)),
            pl.BlockSpec((D_FEAT, D_FEAT), lambda i: (0, 0)),
        ],
        out_specs=pl.BlockSpec((blk, D_FEAT), lambda i: (i, 0)),
        out_shape=jax.ShapeDtypeStruct((N_NODES, D_FEAT), jnp.float32),
    )(node_feats, W_up)


def _q_body(ef_ref, ea_ref, w1_ref, w2_ref, w3_ref, w4_ref, o_ref):
    t = jax.nn.silu(
        lax.dot_general(ef_ref[...], w1_ref[...], (((0,), (0,)), ((), ())),
                        preferred_element_type=jnp.float32) * (1.0 / math.sqrt(D_RADIAL)))
    t = jax.nn.silu(jnp.dot(t, w2_ref[...],
                            preferred_element_type=jnp.float32) * (1.0 / math.sqrt(D_HID)))
    t = jax.nn.silu(jnp.dot(t, w3_ref[...],
                            preferred_element_type=jnp.float32) * (1.0 / math.sqrt(D_HID)))
    w = jnp.dot(t, w4_ref[...], preferred_element_type=jnp.float32) * (1.0 / math.sqrt(D_HID))
    o_ref[...] = w * lax.transpose(ea_ref[...], (1, 0))


def _compute_q(ef_t, ea_t, W1, W2, W3, W4, col0, nrows):
    blk = 6400
    grid = nrows // blk
    off = col0 // blk
    return pl.pallas_call(
        _q_body,
        grid=(grid,),
        in_specs=[
            pl.BlockSpec((D_RADIAL, blk), lambda i: (0, i + off)),
            pl.BlockSpec((1, blk), lambda i: (0, i + off)),
            pl.BlockSpec((D_RADIAL, D_HID), lambda i: (0, 0)),
            pl.BlockSpec((D_HID, D_HID), lambda i: (0, 0)),
            pl.BlockSpec((D_HID, D_HID), lambda i: (0, 0)),
            pl.BlockSpec((D_HID, D_FEAT), lambda i: (0, 0)),
        ],
        out_specs=pl.BlockSpec((blk, D_FEAT), lambda i: (i, 0)),
        out_shape=jax.ShapeDtypeStruct((nrows, D_FEAT), jnp.float32),
    )(ef_t, ea_t, W1, W2, W3, W4)


def _make_sc_kernel(ept, steps, c):
    mesh = plsc.VectorSubcoreMesh(core_axis_name="c", subcore_axis_name="s")

    def body_fn(idx_hbm, q_hbm, h_hbm, out_hbm,
                idx_v, q_v, g_v, acc, isem0, isem1, isem2, isem3,
                qsem, gsem, ssem):
        ci = lax.axis_index("c")
        s = lax.axis_index("s")
        wid = s * _NC + ci
        isems = (isem0, isem1, isem2, isem3)

        def idxload(k, ib):
            pltpu.async_copy(idx_hbm.at[wid, k], idx_v.at[ib], isems[ib])

        def wait_idx(ib):
            pltpu.make_async_copy(idx_hbm.at[0, 0], idx_v.at[ib], isems[ib]).wait()

        def qgload(k, qb, ib):
            base = wid * ept + k * c
            pltpu.async_copy(q_hbm.at[pl.ds(base, c), :], q_v.at[qb], qsem)
            pltpu.async_copy(h_hbm.at[idx_v.at[ib, 0]], g_v.at[qb], gsem)

        def wait_qg(qb, ib):
            pltpu.make_async_copy(q_hbm.at[pl.ds(0, c), :], q_v.at[qb], qsem).wait()
            pltpu.make_async_copy(h_hbm.at[idx_v.at[ib, 0]], g_v.at[qb], gsem).wait()

        def scatter(qb, ib):
            pltpu.async_copy(q_v.at[qb], acc.at[idx_v.at[ib, 1]], ssem, add=True)

        def wait_scatter(qb, ib):
            pltpu.make_async_copy(q_v.at[qb], acc.at[idx_v.at[ib, 1]], ssem).wait()

        def mul_chunk(qb):
            def mbody(e, _):
                for j in range(D_FEAT // _LANES):
                    sl = pl.ds(j * _LANES, _LANES)
                    q_v[qb, e, sl] = q_v[qb, e, sl] * g_v[qb, e, sl]
                return 0
            lax.fori_loop(0, c, mbody, 0)

        idxload(0, 0)
        idxload(1, 1)
        def zbody(e, _):
            for j in range(D_FEAT // _LANES):
                g_v[0, e, pl.ds(j * _LANES, _LANES)] = jnp.zeros((_LANES,), jnp.float32)
            return 0
        lax.fori_loop(0, c, zbody, 0)
        row0 = s * _RPT
        for m in range(_RPT // c):
            pltpu.sync_copy(g_v.at[0], acc.at[pl.ds(row0 + m * c, c), :])
        plsc.subcore_barrier()

        wait_idx(0)
        qgload(0, 0, 0)

        def body(k, qb, ib):
            wait_qg(qb, ib)
            lax.cond(k + 2 < steps,
                     lambda: idxload(k + 2, (ib + 2) % 4), lambda: None)

            def prep_next():
                lax.cond(k >= 1, lambda: wait_scatter(1 - qb, (ib + 3) % 4),
                         lambda: None)
                wait_idx((ib + 1) % 4)
                qgload(k + 1, 1 - qb, (ib + 1) % 4)
            lax.cond(k + 1 < steps, prep_next, lambda: None)

            mul_chunk(qb)
            scatter(qb, ib)

        def step4(i, _):
            for m in range(4):
                body(4 * i + m, m % 2, m)
            return 0
        lax.fori_loop(0, steps // 4, step4, 0)
        for t in range(steps - (steps // 4) * 4):
            kt = (steps // 4) * 4 + t
            body(kt, kt % 2, kt % 4)
        wait_scatter(0, 0)
        wait_scatter(0, 0)

        plsc.subcore_barrier()

        for m in range(_RPT // c):
            r0 = row0 + m * c
            pltpu.sync_copy(acc.at[pl.ds(r0, c), :], g_v.at[0])
            pltpu.sync_copy(g_v.at[0], out_hbm.at[ci, pl.ds(r0, c), :])

    return pl.kernel(
        body_fn,
        out_type=jax.ShapeDtypeStruct((_NC, _NPAD, D_FEAT), jnp.float32),
        mesh=mesh,
        scratch_types=[
            pltpu.VMEM((4, 2, c), jnp.int32),
            pltpu.VMEM((2, c, D_FEAT), jnp.float32),
            pltpu.VMEM((2, c, D_FEAT), jnp.float32),
            pltpu.VMEM_SHARED((_NPAD, D_FEAT), jnp.float32),
            pltpu.SemaphoreType.DMA,
            pltpu.SemaphoreType.DMA,
            pltpu.SemaphoreType.DMA,
            pltpu.SemaphoreType.DMA,
            pltpu.SemaphoreType.DMA,
            pltpu.SemaphoreType.DMA,
            pltpu.SemaphoreType.DMA,
        ],
    )


def _scatter_gather(sender, receiver, q, h, ept, steps, c):
    f = _make_sc_kernel(ept, steps, c)
    idx = jnp.stack([sender.reshape(_NC * _NS, steps, c),
                     receiver.reshape(_NC * _NS, steps, c)], axis=2)
    return f(idx, q, h)


def _fin_body(pa_ref, pb_ref, w_ref, o_ref):
    scale = 1.0 / math.sqrt(D_FEAT) / AVG_NUM_NEIGHBORS
    msg = pa_ref[0] + pa_ref[1] + pb_ref[0] + pb_ref[1]
    o_ref[...] = jnp.dot(msg, w_ref[...], preferred_element_type=jnp.float32) * scale


def _finalize(pa, pb, W_lin):
    blk = 1000
    grid = N_NODES // blk
    return pl.pallas_call(
        _fin_body,
        grid=(grid,),
        in_specs=[
            pl.BlockSpec((_NC, blk, D_FEAT), lambda i: (0, i, 0)),
            pl.BlockSpec((_NC, blk, D_FEAT), lambda i: (0, i, 0)),
            pl.BlockSpec((D_FEAT, D_FEAT), lambda i: (0, 0)),
        ],
        out_specs=pl.BlockSpec((blk, D_FEAT), lambda i: (i, 0)),
        out_shape=jax.ShapeDtypeStruct((N_NODES, D_FEAT), jnp.float32),
    )(pa, pb, W_lin)


def kernel(node_attrs, node_feats, edge_attrs, edge_feats, edge_index,
           W_up, W1, W2, W3, W4, W_lin, W_skip):
    sender = edge_index[0].astype(jnp.int32)
    receiver = edge_index[1].astype(jnp.int32)
    ef_t = edge_feats.T
    ea_t = edge_attrs.reshape(1, N_EDGES)
    ept = N_EDGES // (_NC * _NS)
    c2 = 80
    steps = ept // c2
    h = _compute_h(node_feats, W_up)
    q = _compute_q(ef_t, ea_t, W1, W2, W3, W4, 0, N_EDGES)
    p_a = _scatter_gather(sender, receiver, q, h, ept, steps, c2)
    return _finalize(p_a, p_a * 0.0, W_lin)


# speedup vs baseline: 1.0259x; 1.0229x over previous
import functools
import math

import jax
import jax.numpy as jnp
from jax import lax
from jax.experimental import pallas as pl
from jax.experimental.pallas import tpu as pltpu
from jax.experimental.pallas import tpu_sc as plsc

N_NODES = 10000
N_EDGES = 320000
D_FEAT = 128
D_ATTR = 4
D_RADIAL = 8
D_HID = 64
AVG_NUM_NEIGHBORS = 32.0

_NC = 2
_NS = 16
_LANES = 16

_C = 80
_EPT = N_EDGES // (_NC * _NS)
_STEPS = _EPT // _C
_NPAD = 10240
_RPT = _NPAD // _NS


def _h_body(nf_ref, w_ref, o_ref):
    o_ref[...] = jnp.dot(nf_ref[...], w_ref[...],
                         preferred_element_type=jnp.float32) * (1.0 / math.sqrt(D_FEAT))


def _compute_h(node_feats, W_up):
    blk = 2000
    grid = N_NODES // blk
    return pl.pallas_call(
        _h_body,
        grid=(grid,),
        in_specs=[
            pl.BlockSpec((blk, D_FEAT), lambda i: (i, 0)),
            pl.BlockSpec((D_FEAT, D_FEAT), lambda i: (0, 0)),
        ],
        out_specs=pl.BlockSpec((blk, D_FEAT), lambda i: (i, 0)),
        out_shape=jax.ShapeDtypeStruct((N_NODES, D_FEAT), jnp.float32),
    )(node_feats, W_up)


def _q_body(ef_ref, ea_ref, w1_ref, w2_ref, w3_ref, w4_ref, o_ref):
    t = jax.nn.silu(
        lax.dot_general(ef_ref[...], w1_ref[...], (((0,), (0,)), ((), ())),
                        preferred_element_type=jnp.float32) * (1.0 / math.sqrt(D_RADIAL)))
    t = jax.nn.silu(jnp.dot(t, w2_ref[...],
                            preferred_element_type=jnp.float32) * (1.0 / math.sqrt(D_HID)))
    t = jax.nn.silu(jnp.dot(t, w3_ref[...],
                            preferred_element_type=jnp.float32) * (1.0 / math.sqrt(D_HID)))
    w = jnp.dot(t, w4_ref[...], preferred_element_type=jnp.float32) * (1.0 / math.sqrt(D_HID))
    o_ref[...] = w * lax.transpose(ea_ref[...], (1, 0))


def _compute_q(edge_feats, edge_attrs, W1, W2, W3, W4):
    blk = 12800
    grid = N_EDGES // blk
    ef_t = edge_feats.T
    ea_t = edge_attrs.reshape(1, N_EDGES)
    return pl.pallas_call(
        _q_body,
        grid=(grid,),
        in_specs=[
            pl.BlockSpec((D_RADIAL, blk), lambda i: (0, i)),
            pl.BlockSpec((1, blk), lambda i: (0, i)),
            pl.BlockSpec((D_RADIAL, D_HID), lambda i: (0, 0)),
            pl.BlockSpec((D_HID, D_HID), lambda i: (0, 0)),
            pl.BlockSpec((D_HID, D_HID), lambda i: (0, 0)),
            pl.BlockSpec((D_HID, D_FEAT), lambda i: (0, 0)),
        ],
        out_specs=pl.BlockSpec((blk, D_FEAT), lambda i: (i, 0)),
        out_shape=jax.ShapeDtypeStruct((N_EDGES, D_FEAT), jnp.float32),
    )(ef_t, ea_t, W1, W2, W3, W4)


def _mul_chunk(q_v, g_v):
    def mbody(e, _):
        for j in range(D_FEAT // _LANES):
            sl = pl.ds(j * _LANES, _LANES)
            q_v[e, sl] = q_v[e, sl] * g_v[e, sl]
        return 0
    lax.fori_loop(0, _C, mbody, 0)


def _sc_body(idx_hbm, q_hbm, h_hbm, out_hbm,
             idx_v, ridx_v, q_v, g_v, acc, isem0, isem1, qsem, gsem, ssem):
    c = lax.axis_index("c")
    s = lax.axis_index("s")
    wid = s * _NC + c
    isems = (isem0, isem1)

    def idxload(k, b):
        pltpu.async_copy(idx_hbm.at[wid, k], idx_v.at[b], isems[b])

    def wait_idx(b):
        pltpu.make_async_copy(idx_hbm.at[0, 0], idx_v.at[b], isems[b]).wait()

    def qgload(k, b):
        base = wid * _EPT + k * _C
        pltpu.async_copy(q_hbm.at[pl.ds(base, _C), :], q_v.at[b], qsem)
        pltpu.async_copy(h_hbm.at[idx_v.at[b, 0]], g_v.at[b], gsem)

    def wait_qg(b):
        pltpu.make_async_copy(q_hbm.at[pl.ds(0, _C), :], q_v.at[b], qsem).wait()
        pltpu.make_async_copy(h_hbm.at[idx_v.at[b, 0]], g_v.at[b], gsem).wait()

    def scatter(k, b):
        pltpu.async_copy(q_v.at[b], acc.at[ridx_v.at[b]], ssem, add=True)

    def wait_scatter(b):
        pltpu.make_async_copy(q_v.at[b], acc.at[ridx_v.at[b]], ssem).wait()

    idxload(0, 0)
    idxload(1, 1)

    def zbody(e, _):
        for j in range(D_FEAT // _LANES):
            g_v[0, e, pl.ds(j * _LANES, _LANES)] = jnp.zeros((_LANES,), jnp.float32)
        return 0
    lax.fori_loop(0, _C, zbody, 0)
    row0 = s * _RPT
    for m in range(_RPT // _C):
        pltpu.sync_copy(g_v.at[0], acc.at[pl.ds(row0 + m * _C, _C), :])
    plsc.subcore_barrier()

    wait_idx(0)
    qgload(0, 0)

    def body(k, b):
        wait_qg(b)
        for j in range(_C // _LANES):
            sl = pl.ds(j * _LANES, _LANES)
            ridx_v[b, sl] = idx_v[b, 1, sl]
        lax.cond(k + 2 < _STEPS, lambda: idxload(k + 2, b), lambda: None)

        def prep_next():
            lax.cond(k >= 1, lambda: wait_scatter(1 - b), lambda: None)
            wait_idx(1 - b)
            qgload(k + 1, 1 - b)
        lax.cond(k + 1 < _STEPS, prep_next, lambda: None)

        _mul_chunk(q_v.at[b], g_v.at[b])
        scatter(k, b)

    def step2(i, _):
        body(2 * i, 0)
        body(2 * i + 1, 1)
        return 0
    lax.fori_loop(0, _STEPS // 2, step2, 0)
    body(_STEPS - 1, 0)
    wait_scatter(1)
    wait_scatter(0)

    plsc.subcore_barrier()

    for m in range(_RPT // _C):
        r0 = row0 + m * _C
        pltpu.sync_copy(acc.at[pl.ds(r0, _C), :], g_v.at[0])
        pltpu.sync_copy(g_v.at[0], out_hbm.at[c, pl.ds(r0, _C), :])


def _scatter_gather(sender, receiver, q, h):
    mesh = plsc.VectorSubcoreMesh(core_axis_name="c", subcore_axis_name="s")
    f = pl.kernel(
        _sc_body,
        out_type=jax.ShapeDtypeStruct((_NC, _NPAD, D_FEAT), jnp.float32),
        mesh=mesh,
        scratch_types=[
            pltpu.VMEM((2, 2, _C), jnp.int32),
            pltpu.VMEM((2, _C), jnp.int32),
            pltpu.VMEM((2, _C, D_FEAT), jnp.float32),
            pltpu.VMEM((2, _C, D_FEAT), jnp.float32),
            pltpu.VMEM_SHARED((_NPAD, D_FEAT), jnp.float32),
            pltpu.SemaphoreType.DMA,
            pltpu.SemaphoreType.DMA,
            pltpu.SemaphoreType.DMA,
            pltpu.SemaphoreType.DMA,
            pltpu.SemaphoreType.DMA,
        ],
    )
    idx = jnp.stack([sender.reshape(_NC * _NS, _STEPS, _C),
                     receiver.reshape(_NC * _NS, _STEPS, _C)], axis=2)
    return f(idx, q, h)


def _fin_body(p_ref, w_ref, o_ref):
    scale = 1.0 / math.sqrt(D_FEAT) / AVG_NUM_NEIGHBORS
    msg = p_ref[0] + p_ref[1]
    o_ref[...] = jnp.dot(msg, w_ref[...], preferred_element_type=jnp.float32) * scale


def _finalize(partials, W_lin):
    blk = 1000
    grid = N_NODES // blk
    return pl.pallas_call(
        _fin_body,
        grid=(grid,),
        in_specs=[
            pl.BlockSpec((_NC, blk, D_FEAT), lambda i: (0, i, 0)),
            pl.BlockSpec((D_FEAT, D_FEAT), lambda i: (0, 0)),
        ],
        out_specs=pl.BlockSpec((blk, D_FEAT), lambda i: (i, 0)),
        out_shape=jax.ShapeDtypeStruct((N_NODES, D_FEAT), jnp.float32),
    )(partials, W_lin)


def kernel(node_attrs, node_feats, edge_attrs, edge_feats, edge_index,
           W_up, W1, W2, W3, W4, W_lin, W_skip):
    sender = edge_index[0].astype(jnp.int32)
    receiver = edge_index[1].astype(jnp.int32)
    h = _compute_h(node_feats, W_up)
    q = _compute_q(edge_feats, edge_attrs, W1, W2, W3, W4)
    partials = _scatter_gather(sender, receiver, q, h)
    return _finalize(partials, W_lin)


# speedup vs baseline: 1.0285x; 1.0025x over previous
import functools
import math

import jax
import jax.numpy as jnp
from jax import lax
from jax.experimental import pallas as pl
from jax.experimental.pallas import tpu as pltpu
from jax.experimental.pallas import tpu_sc as plsc

N_NODES = 10000
N_EDGES = 320000
D_FEAT = 128
D_ATTR = 4
D_RADIAL = 8
D_HID = 64
AVG_NUM_NEIGHBORS = 32.0

_NC = 2
_NS = 16
_LANES = 16

_C = 80
_EPT = N_EDGES // (_NC * _NS)
_STEPS = _EPT // _C
_NPAD = 10240
_RPT = _NPAD // _NS


def _h_body(nf_ref, w_ref, o_ref):
    o_ref[...] = jnp.dot(nf_ref[...], w_ref[...],
                         preferred_element_type=jnp.float32) * (1.0 / math.sqrt(D_FEAT))


def _compute_h(node_feats, W_up):
    blk = 2000
    grid = N_NODES // blk
    return pl.pallas_call(
        _h_body,
        grid=(grid,),
        in_specs=[
            pl.BlockSpec((blk, D_FEAT), lambda i: (i, 0)),
            pl.BlockSpec((D_FEAT, D_FEAT), lambda i: (0, 0)),
        ],
        out_specs=pl.BlockSpec((blk, D_FEAT), lambda i: (i, 0)),
        out_shape=jax.ShapeDtypeStruct((N_NODES, D_FEAT), jnp.float32),
    )(node_feats, W_up)


def _q_body(ef_ref, ea_ref, w1_ref, w2_ref, w3_ref, w4_ref, o_ref):
    t = jax.nn.silu(
        lax.dot_general(ef_ref[...], w1_ref[...], (((0,), (0,)), ((), ())),
                        preferred_element_type=jnp.float32) * (1.0 / math.sqrt(D_RADIAL)))
    t = jax.nn.silu(jnp.dot(t, w2_ref[...],
                            preferred_element_type=jnp.float32) * (1.0 / math.sqrt(D_HID)))
    t = jax.nn.silu(jnp.dot(t, w3_ref[...],
                            preferred_element_type=jnp.float32) * (1.0 / math.sqrt(D_HID)))
    w = jnp.dot(t, w4_ref[...], preferred_element_type=jnp.float32) * (1.0 / math.sqrt(D_HID))
    o_ref[...] = w * lax.transpose(ea_ref[...], (1, 0))


def _compute_q(edge_feats, edge_attrs, W1, W2, W3, W4):
    blk = 12800
    grid = N_EDGES // blk
    ef_t = edge_feats.T
    ea_t = edge_attrs.reshape(1, N_EDGES)
    return pl.pallas_call(
        _q_body,
        grid=(grid,),
        in_specs=[
            pl.BlockSpec((D_RADIAL, blk), lambda i: (0, i)),
            pl.BlockSpec((1, blk), lambda i: (0, i)),
            pl.BlockSpec((D_RADIAL, D_HID), lambda i: (0, 0)),
            pl.BlockSpec((D_HID, D_HID), lambda i: (0, 0)),
            pl.BlockSpec((D_HID, D_HID), lambda i: (0, 0)),
            pl.BlockSpec((D_HID, D_FEAT), lambda i: (0, 0)),
        ],
        out_specs=pl.BlockSpec((blk, D_FEAT), lambda i: (i, 0)),
        out_shape=jax.ShapeDtypeStruct((N_EDGES, D_FEAT), jnp.float32),
    )(ef_t, ea_t, W1, W2, W3, W4)


def _mul_chunk(q_v, g_v):
    def mbody(i, _):
        for u in range(4):
            e = 4 * i + u
            for j in range(D_FEAT // _LANES):
                sl = pl.ds(j * _LANES, _LANES)
                q_v[e, sl] = q_v[e, sl] * g_v[e, sl]
        return 0
    lax.fori_loop(0, _C // 4, mbody, 0)


def _sc_body(idx_hbm, q_hbm, h_hbm, out_hbm,
             idx_v, ridx_v, q_v, g_v, acc, isem0, isem1, qsem, gsem, ssem):
    c = lax.axis_index("c")
    s = lax.axis_index("s")
    wid = s * _NC + c
    isems = (isem0, isem1)

    def idxload(k, b):
        pltpu.async_copy(idx_hbm.at[wid, k], idx_v.at[b], isems[b])

    def wait_idx(b):
        pltpu.make_async_copy(idx_hbm.at[0, 0], idx_v.at[b], isems[b]).wait()

    def qgload(k, b):
        base = wid * _EPT + k * _C
        pltpu.async_copy(q_hbm.at[pl.ds(base, _C), :], q_v.at[b], qsem)
        pltpu.async_copy(h_hbm.at[idx_v.at[b, 0]], g_v.at[b], gsem)

    def wait_qg(b):
        pltpu.make_async_copy(q_hbm.at[pl.ds(0, _C), :], q_v.at[b], qsem).wait()
        pltpu.make_async_copy(h_hbm.at[idx_v.at[b, 0]], g_v.at[b], gsem).wait()

    def scatter(k, b):
        pltpu.async_copy(q_v.at[b], acc.at[ridx_v.at[b]], ssem, add=True)

    def wait_scatter(b):
        pltpu.make_async_copy(q_v.at[b], acc.at[ridx_v.at[b]], ssem).wait()

    idxload(0, 0)
    idxload(1, 1)

    def zbody(e, _):
        for j in range(D_FEAT // _LANES):
            g_v[0, e, pl.ds(j * _LANES, _LANES)] = jnp.zeros((_LANES,), jnp.float32)
        return 0
    lax.fori_loop(0, _C, zbody, 0)
    row0 = s * _RPT
    for m in range(_RPT // _C):
        pltpu.sync_copy(g_v.at[0], acc.at[pl.ds(row0 + m * _C, _C), :])
    plsc.subcore_barrier()

    wait_idx(0)
    qgload(0, 0)

    def body(k, b):
        wait_qg(b)
        for j in range(_C // _LANES):
            sl = pl.ds(j * _LANES, _LANES)
            ridx_v[b, sl] = idx_v[b, 1, sl]
        lax.cond(k + 2 < _STEPS, lambda: idxload(k + 2, b), lambda: None)

        def prep_next():
            lax.cond(k >= 1, lambda: wait_scatter(1 - b), lambda: None)
            wait_idx(1 - b)
            qgload(k + 1, 1 - b)
        lax.cond(k + 1 < _STEPS, prep_next, lambda: None)

        _mul_chunk(q_v.at[b], g_v.at[b])
        scatter(k, b)

    def step2(i, _):
        body(2 * i, 0)
        body(2 * i + 1, 1)
        return 0
    lax.fori_loop(0, _STEPS // 2, step2, 0)
    body(_STEPS - 1, 0)
    wait_scatter(1)
    wait_scatter(0)

    plsc.subcore_barrier()

    for m in range(_RPT // _C):
        r0 = row0 + m * _C
        pltpu.sync_copy(acc.at[pl.ds(r0, _C), :], g_v.at[0])
        pltpu.sync_copy(g_v.at[0], out_hbm.at[c, pl.ds(r0, _C), :])


def _scatter_gather(sender, receiver, q, h):
    mesh = plsc.VectorSubcoreMesh(core_axis_name="c", subcore_axis_name="s")
    f = pl.kernel(
        _sc_body,
        out_type=jax.ShapeDtypeStruct((_NC, _NPAD, D_FEAT), jnp.float32),
        mesh=mesh,
        scratch_types=[
            pltpu.VMEM((2, 2, _C), jnp.int32),
            pltpu.VMEM((2, _C), jnp.int32),
            pltpu.VMEM((2, _C, D_FEAT), jnp.float32),
            pltpu.VMEM((2, _C, D_FEAT), jnp.float32),
            pltpu.VMEM_SHARED((_NPAD, D_FEAT), jnp.float32),
            pltpu.SemaphoreType.DMA,
            pltpu.SemaphoreType.DMA,
            pltpu.SemaphoreType.DMA,
            pltpu.SemaphoreType.DMA,
            pltpu.SemaphoreType.DMA,
        ],
    )
    idx = jnp.stack([sender.reshape(_NC * _NS, _STEPS, _C),
                     receiver.reshape(_NC * _NS, _STEPS, _C)], axis=2)
    return f(idx, q, h)


def _fin_body(p_ref, w_ref, o_ref):
    scale = 1.0 / math.sqrt(D_FEAT) / AVG_NUM_NEIGHBORS
    msg = p_ref[0] + p_ref[1]
    o_ref[...] = jnp.dot(msg, w_ref[...], preferred_element_type=jnp.float32) * scale


def _finalize(partials, W_lin):
    blk = 1000
    grid = N_NODES // blk
    return pl.pallas_call(
        _fin_body,
        grid=(grid,),
        in_specs=[
            pl.BlockSpec((_NC, blk, D_FEAT), lambda i: (0, i, 0)),
            pl.BlockSpec((D_FEAT, D_FEAT), lambda i: (0, 0)),
        ],
        out_specs=pl.BlockSpec((blk, D_FEAT), lambda i: (i, 0)),
        out_shape=jax.ShapeDtypeStruct((N_NODES, D_FEAT), jnp.float32),
    )(partials, W_lin)


def kernel(node_attrs, node_feats, edge_attrs, edge_feats, edge_index,
           W_up, W1, W2, W3, W4, W_lin, W_skip):
    sender = edge_index[0].astype(jnp.int32)
    receiver = edge_index[1].astype(jnp.int32)
    h = _compute_h(node_feats, W_up)
    q = _compute_q(edge_feats, edge_attrs, W1, W2, W3, W4)
    partials = _scatter_gather(sender, receiver, q, h)
    return _finalize(partials, W_lin)
